# Initial kernel scaffold; baseline (speedup 1.0000x reference)
import jax
import jax.numpy as jnp
from jax.experimental import pallas as pl


def kernel(x, edge_index, W1, b1, W2, b2):
    raise NotImplementedError("write your pallas kernel here")


# speedup vs baseline: 21.0482x; 21.0482x over previous
import functools

import jax
import jax.numpy as jnp
from jax import lax
from jax.experimental import pallas as pl
from jax.experimental.pallas import tpu as pltpu
from jax.experimental.pallas import tpu_sc as plsc

N = 10000
E = 320000
IN_CH = 128
HID_CH = 64

NC = 2
NS = 16
NW = NC * NS
CHUNK = 128
CPT = 80
EP = NW * CPT * CHUNK
NP = 10240
RPT = NP // NS

_MESH = plsc.VectorSubcoreMesh(
    core_axis_name="c", subcore_axis_name="s", num_cores=NC, num_subcores=NS
)


def _worker(cid, sid):
    return cid * NS + sid


@functools.partial(
    pl.kernel,
    out_type=jax.ShapeDtypeStruct((NC, NP), jnp.float32),
    mesh=_MESH,
    scratch_types=[
        pltpu.VMEM((CPT, CHUNK), jnp.int32),
        pltpu.VMEM((CHUNK,), jnp.float32),
        pltpu.VMEM((RPT,), jnp.float32),
        pltpu.VMEM_SHARED((NP,), jnp.float32),
        pltpu.SemaphoreType.DMA,
        pltpu.SemaphoreType.DMA,
        pltpu.SemaphoreType.DMA,
    ],
)
def _sc_degree(dstI_hbm, out_hbm, didx, ones, zbuf, acc, isem, s0, s1):
    cid = lax.axis_index("c")
    sid = lax.axis_index("s")
    wid = _worker(cid, sid)
    cp = pltpu.async_copy(dstI_hbm.at[pl.ds(wid * CPT, CPT)], didx, isem)

    for j in range(CHUNK // 16):
        ones[pl.ds(j * 16, 16)] = jnp.full((16,), 1.0, jnp.float32)

    def zrow(i, carry):
        zbuf[pl.ds(i * 16, 16)] = jnp.zeros((16,), jnp.float32)
        return carry

    lax.fori_loop(0, RPT // 16, zrow, 0)
    pltpu.sync_copy(zbuf, acc.at[pl.ds(sid * RPT, RPT)])
    cp.wait()
    plsc.subcore_barrier()

    def body(t, carry):
        c0 = 2 * t
        d0 = pltpu.async_copy(ones, acc.at[didx.at[c0]], s0, add=True)
        d1 = pltpu.async_copy(ones, acc.at[didx.at[c0 + 1]], s1, add=True)
        d0.wait()
        d1.wait()
        return carry

    lax.fori_loop(0, CPT // 2, body, 0)
    plsc.subcore_barrier()
    pltpu.sync_copy(acc.at[pl.ds(sid * RPT, RPT)],
                    out_hbm.at[cid, pl.ds(sid * RPT, RPT)])


@functools.partial(
    pl.kernel,
    out_type=jax.ShapeDtypeStruct((NC, NP, HID_CH), jnp.float32),
    mesh=_MESH,
    compiler_params=pltpu.CompilerParams(use_tc_tiling_on_sc=False),
    scratch_types=[
        pltpu.VMEM((CPT, CHUNK), jnp.int32),
        pltpu.VMEM((CPT, CHUNK), jnp.int32),
        pltpu.VMEM((CHUNK, HID_CH), jnp.float32),
        pltpu.VMEM((CHUNK, HID_CH), jnp.float32),
        pltpu.VMEM((CHUNK, HID_CH), jnp.float32),
        pltpu.VMEM_SHARED((NP, HID_CH), jnp.float32),
        pltpu.SemaphoreType.DMA,
        pltpu.SemaphoreType.DMA,
        pltpu.SemaphoreType.DMA,
        pltpu.SemaphoreType.DMA,
        pltpu.SemaphoreType.DMA,
    ],
)
def _sc_agg_rows(y_hbm, srcI_hbm, dstI_hbm, out_hbm, sidx, didx, rows0, rows1,
                 zbuf, acc, isem, g0, g1, s0, s1):
    cid = lax.axis_index("c")
    sid = lax.axis_index("s")
    wid = _worker(cid, sid)
    cps = pltpu.async_copy(srcI_hbm.at[pl.ds(wid * CPT, CPT)], sidx, isem)
    cpd = pltpu.async_copy(dstI_hbm.at[pl.ds(wid * CPT, CPT)], didx, isem)

    def zrow(i, carry):
        for j in range(HID_CH // 16):
            zbuf[i, pl.ds(j * 16, 16)] = jnp.zeros((16,), jnp.float32)
        return carry

    lax.fori_loop(0, CHUNK, zrow, 0)
    for k in range(RPT // CHUNK):
        pltpu.sync_copy(zbuf, acc.at[pl.ds(sid * RPT + k * CHUNK, CHUNK)])
    cps.wait()
    cpd.wait()
    plsc.subcore_barrier()

    def body(t, carry):
        c0 = 2 * t
        pltpu.async_copy(y_hbm.at[sidx.at[c0]], rows0, g0).wait()
        d0 = pltpu.async_copy(rows0, acc.at[didx.at[c0]], s0, add=True)
        pltpu.async_copy(y_hbm.at[sidx.at[c0 + 1]], rows1, g1).wait()
        d1 = pltpu.async_copy(rows1, acc.at[didx.at[c0 + 1]], s1, add=True)
        d0.wait()
        d1.wait()
        return carry

    lax.fori_loop(0, CPT // 2, body, 0)
    plsc.subcore_barrier()
    pltpu.sync_copy(acc.at[pl.ds(sid * RPT, RPT)],
                    out_hbm.at[cid, pl.ds(sid * RPT, RPT)])


@functools.partial(
    pl.kernel,
    out_type=jax.ShapeDtypeStruct((NC, NP), jnp.float32),
    mesh=_MESH,
    scratch_types=[
        pltpu.VMEM((CPT, CHUNK), jnp.int32),
        pltpu.VMEM((CPT, CHUNK), jnp.int32),
        pltpu.VMEM((CHUNK,), jnp.float32),
        pltpu.VMEM((CHUNK,), jnp.float32),
        pltpu.VMEM((RPT,), jnp.float32),
        pltpu.VMEM_SHARED((NP,), jnp.float32),
        pltpu.SemaphoreType.DMA,
        pltpu.SemaphoreType.DMA,
        pltpu.SemaphoreType.DMA,
        pltpu.SemaphoreType.DMA,
        pltpu.SemaphoreType.DMA,
    ],
)
def _sc_agg_scalar(z_hbm, srcI_hbm, dstI_hbm, out_hbm, sidx, didx, vals0,
                   vals1, zbuf, acc, isem, g0, g1, s0, s1):
    cid = lax.axis_index("c")
    sid = lax.axis_index("s")
    wid = _worker(cid, sid)
    cps = pltpu.async_copy(srcI_hbm.at[pl.ds(wid * CPT, CPT)], sidx, isem)
    cpd = pltpu.async_copy(dstI_hbm.at[pl.ds(wid * CPT, CPT)], didx, isem)

    def zrow(i, carry):
        zbuf[pl.ds(i * 16, 16)] = jnp.zeros((16,), jnp.float32)
        return carry

    lax.fori_loop(0, RPT // 16, zrow, 0)
    pltpu.sync_copy(zbuf, acc.at[pl.ds(sid * RPT, RPT)])
    cps.wait()
    cpd.wait()
    plsc.subcore_barrier()

    def body(t, carry):
        c0 = 2 * t
        pltpu.async_copy(z_hbm.at[sidx.at[c0]], vals0, g0).wait()
        d0 = pltpu.async_copy(vals0, acc.at[didx.at[c0]], s0, add=True)
        pltpu.async_copy(z_hbm.at[sidx.at[c0 + 1]], vals1, g1).wait()
        d1 = pltpu.async_copy(vals1, acc.at[didx.at[c0 + 1]], s1, add=True)
        d0.wait()
        d1.wait()
        return carry

    lax.fori_loop(0, CPT // 2, body, 0)
    plsc.subcore_barrier()
    pltpu.sync_copy(acc.at[pl.ds(sid * RPT, RPT)],
                    out_hbm.at[cid, pl.ds(sid * RPT, RPT)])


def _tc_m1_body(x_ref, w1_ref, degp_ref, xw_ref, y_ref, dinv_ref):
    deg = degp_ref[0] + degp_ref[1] + 1.0
    dinv = lax.rsqrt(deg)
    xw = jnp.dot(x_ref[...], w1_ref[...], preferred_element_type=jnp.float32)
    xw_ref[...] = xw
    y_ref[...] = xw * dinv
    dinv_ref[...] = dinv


_tc_m1 = pl.pallas_call(
    _tc_m1_body,
    out_shape=[
        jax.ShapeDtypeStruct((NP, HID_CH), jnp.float32),
        jax.ShapeDtypeStruct((NP, HID_CH), jnp.float32),
        jax.ShapeDtypeStruct((NP, 1), jnp.float32),
    ],
)


def _tc_mid_body(p_ref, xw_ref, dinv_ref, b1_ref, w2t_ref, b2_ref,
                 z_ref, self2_ref):
    dinv = dinv_ref[...]
    agg = p_ref[0] + p_ref[1]
    h = jnp.maximum(dinv * agg + (dinv * dinv) * xw_ref[...] + b1_ref[...],
                    0.0)
    hw = jnp.sum(h * w2t_ref[...], axis=1, keepdims=True)
    z_ref[...] = dinv * hw
    self2_ref[...] = (dinv * dinv) * hw + b2_ref[...]


_tc_mid = pl.pallas_call(
    _tc_mid_body,
    out_shape=[
        jax.ShapeDtypeStruct((NP, 1), jnp.float32),
        jax.ShapeDtypeStruct((NP, 1), jnp.float32),
    ],
)


def _tc_fin_body(q_ref, dinv_ref, self2_ref, out_ref):
    out_ref[...] = dinv_ref[...] * (q_ref[0] + q_ref[1]) + self2_ref[...]


_tc_fin = pl.pallas_call(
    _tc_fin_body,
    out_shape=jax.ShapeDtypeStruct((NP, 1), jnp.float32),
)


@jax.jit
def kernel(x, edge_index, W1, b1, W2, b2):
    src = edge_index[0].astype(jnp.int32)
    dst = edge_index[1].astype(jnp.int32)
    pad = jnp.full((EP - E,), N, dtype=jnp.int32)
    srcI = jnp.concatenate([src, pad]).reshape(EP // CHUNK, CHUNK)
    dstI = jnp.concatenate([dst, pad]).reshape(EP // CHUNK, CHUNK)
    x_pad = jnp.pad(x, ((0, NP - N), (0, 0)))

    degp = _sc_degree(dstI)
    xw, y, dinv = _tc_m1(x_pad, W1, degp.reshape(NC, NP, 1))
    p1 = _sc_agg_rows(y, srcI, dstI)
    z, self2 = _tc_mid(p1, xw, dinv, b1.reshape(1, HID_CH),
                       W2.reshape(1, HID_CH), b2.reshape(1, 1))
    q = _sc_agg_scalar(z.reshape(NP), srcI, dstI)
    out = _tc_fin(q.reshape(NC, NP, 1), dinv, self2)
    return out[:N, 0]


# speedup vs baseline: 26.5962x; 1.2636x over previous
import functools

import jax
import jax.numpy as jnp
from jax import lax
from jax.experimental import pallas as pl
from jax.experimental.pallas import tpu as pltpu
from jax.experimental.pallas import tpu_sc as plsc

N = 10000
E = 320000
IN_CH = 128
HID_CH = 64

NC = 2
NS = 16
NW = NC * NS
CHUNK = 128
CPT = 80
EPT = CPT * CHUNK
EP = NW * EPT
NP = 10240
RPT = NP // NS
NBUF = 4

_MESH = plsc.VectorSubcoreMesh(
    core_axis_name="c", subcore_axis_name="s", num_cores=NC, num_subcores=NS
)


def _worker(cid, sid):
    return cid * NS + sid


def _zero_1d(ref, n):
    def zrow(i, carry):
        ref[pl.ds(i * 16, 16)] = jnp.zeros((16,), jnp.float32)
        return carry

    lax.fori_loop(0, n // 16, zrow, 0)


@functools.partial(
    pl.kernel,
    out_type=jax.ShapeDtypeStruct((NW, NP), jnp.float32),
    mesh=_MESH,
    compiler_params=pltpu.CompilerParams(needs_layout_passes=False),
    scratch_types=[
        pltpu.VMEM((CPT, CHUNK), jnp.int32),
        pltpu.VMEM((NP,), jnp.float32),
        pltpu.SemaphoreType.DMA,
    ],
)
def _sc_degree(dstI_hbm, out_hbm, didx, accl, isem):
    cid = lax.axis_index("c")
    sid = lax.axis_index("s")
    wid = _worker(cid, sid)
    cp = pltpu.async_copy(dstI_hbm.at[pl.ds(wid * CPT, CPT)], didx, isem)
    _zero_1d(accl, NP)
    cp.wait()
    one = jnp.full((16,), 1.0, jnp.float32)

    def body(r, carry):
        for j in range(CHUNK // 16):
            di = didx[r, pl.ds(j * 16, 16)]
            plsc.addupdate_scatter(accl, [di], one)
        return carry

    lax.fori_loop(0, CPT, body, 0)
    pltpu.sync_copy(accl, out_hbm.at[wid])


@functools.partial(
    pl.kernel,
    out_type=jax.ShapeDtypeStruct((NC, NP, HID_CH), jnp.float32),
    mesh=_MESH,
    compiler_params=pltpu.CompilerParams(use_tc_tiling_on_sc=False),
    scratch_types=[
        pltpu.VMEM((CPT, CHUNK), jnp.int32),
        pltpu.VMEM((CPT, CHUNK), jnp.int32),
        [pltpu.VMEM((CHUNK, HID_CH), jnp.float32)] * NBUF,
        pltpu.VMEM((CHUNK, HID_CH), jnp.float32),
        pltpu.VMEM_SHARED((NP, HID_CH), jnp.float32),
        pltpu.SemaphoreType.DMA,
        [pltpu.SemaphoreType.DMA] * NBUF,
        [pltpu.SemaphoreType.DMA] * NBUF,
    ],
)
def _sc_agg_rows(y_hbm, srcI_hbm, dstI_hbm, out_hbm, sidx, didx, rows,
                 zbuf, acc, isem, gsem, ssem):
    cid = lax.axis_index("c")
    sid = lax.axis_index("s")
    wid = _worker(cid, sid)
    cps = pltpu.async_copy(srcI_hbm.at[pl.ds(wid * CPT, CPT)], sidx, isem)
    cpd = pltpu.async_copy(dstI_hbm.at[pl.ds(wid * CPT, CPT)], didx, isem)

    def zrow(i, carry):
        for j in range(HID_CH // 16):
            zbuf[i, pl.ds(j * 16, 16)] = jnp.zeros((16,), jnp.float32)
        return carry

    lax.fori_loop(0, CHUNK, zrow, 0)
    for k in range(RPT // CHUNK):
        pltpu.sync_copy(zbuf, acc.at[pl.ds(sid * RPT + k * CHUNK, CHUNK)])
    cps.wait()
    cpd.wait()
    plsc.subcore_barrier()

    def body(t, carry):
        c0 = NBUF * t
        gds = []
        for b in range(NBUF):
            @pl.when(t > 0)
            def _drain(b=b):
                pltpu.make_async_copy(rows[b], acc.at[didx.at[0]],
                                      ssem[b]).wait()

            gds.append(
                pltpu.async_copy(y_hbm.at[sidx.at[c0 + b]], rows[b], gsem[b]))
        for b in range(NBUF):
            gds[b].wait()
            pltpu.async_copy(rows[b], acc.at[didx.at[c0 + b]], ssem[b],
                             add=True)
        return carry

    lax.fori_loop(0, CPT // NBUF, body, 0)
    for b in range(NBUF):
        pltpu.make_async_copy(rows[b], acc.at[didx.at[0]], ssem[b]).wait()
    plsc.subcore_barrier()
    pltpu.sync_copy(acc.at[pl.ds(sid * RPT, RPT)],
                    out_hbm.at[cid, pl.ds(sid * RPT, RPT)])


@functools.partial(
    pl.kernel,
    out_type=jax.ShapeDtypeStruct((NW, NP), jnp.float32),
    mesh=_MESH,
    compiler_params=pltpu.CompilerParams(needs_layout_passes=False),
    scratch_types=[
        pltpu.VMEM((CPT, CHUNK), jnp.int32),
        pltpu.VMEM((CPT, CHUNK), jnp.int32),
        pltpu.VMEM((NP,), jnp.float32),
        pltpu.VMEM((NP,), jnp.float32),
        pltpu.SemaphoreType.DMA,
    ],
)
def _sc_agg_scalar(z_hbm, srcI_hbm, dstI_hbm, out_hbm, sidx, didx, zloc,
                   accl, isem):
    cid = lax.axis_index("c")
    sid = lax.axis_index("s")
    wid = _worker(cid, sid)
    cps = pltpu.async_copy(srcI_hbm.at[pl.ds(wid * CPT, CPT)], sidx, isem)
    cpd = pltpu.async_copy(dstI_hbm.at[pl.ds(wid * CPT, CPT)], didx, isem)
    cpz = pltpu.async_copy(z_hbm, zloc, isem)
    _zero_1d(accl, NP)
    cps.wait()
    cpd.wait()
    cpz.wait()

    def body(r, carry):
        for j in range(CHUNK // 16):
            si = sidx[r, pl.ds(j * 16, 16)]
            di = didx[r, pl.ds(j * 16, 16)]
            vals = plsc.load_gather(zloc, [si])
            plsc.addupdate_scatter(accl, [di], vals)
        return carry

    lax.fori_loop(0, CPT, body, 0)
    pltpu.sync_copy(accl, out_hbm.at[wid])


def _tc_m1_body(x_ref, w1_ref, degp_ref, xw_ref, y_ref, dinv_ref):
    deg = jnp.sum(degp_ref[...], axis=1, keepdims=True) + 1.0
    dinv = lax.rsqrt(deg)
    xw = jnp.dot(x_ref[...], w1_ref[...], preferred_element_type=jnp.float32)
    xw_ref[...] = xw
    y_ref[...] = xw * dinv
    dinv_ref[...] = dinv


_tc_m1 = pl.pallas_call(
    _tc_m1_body,
    out_shape=[
        jax.ShapeDtypeStruct((NP, HID_CH), jnp.float32),
        jax.ShapeDtypeStruct((NP, HID_CH), jnp.float32),
        jax.ShapeDtypeStruct((NP, 1), jnp.float32),
    ],
)


def _tc_mid_body(p_ref, xw_ref, dinv_ref, b1_ref, w2t_ref, b2_ref,
                 z_ref, self2_ref):
    dinv = dinv_ref[...]
    agg = p_ref[0] + p_ref[1]
    h = jnp.maximum(dinv * agg + (dinv * dinv) * xw_ref[...] + b1_ref[...],
                    0.0)
    hw = jnp.sum(h * w2t_ref[...], axis=1, keepdims=True)
    z_ref[...] = dinv * hw
    self2_ref[...] = (dinv * dinv) * hw + b2_ref[...]


_tc_mid = pl.pallas_call(
    _tc_mid_body,
    out_shape=[
        jax.ShapeDtypeStruct((NP, 1), jnp.float32),
        jax.ShapeDtypeStruct((NP, 1), jnp.float32),
    ],
)


def _tc_fin_body(q_ref, dinv_ref, self2_ref, out_ref):
    q = jnp.sum(q_ref[...], axis=1, keepdims=True)
    out_ref[...] = dinv_ref[...] * q + self2_ref[...]


_tc_fin = pl.pallas_call(
    _tc_fin_body,
    out_shape=jax.ShapeDtypeStruct((NP, 1), jnp.float32),
)


@jax.jit
def kernel(x, edge_index, W1, b1, W2, b2):
    src = edge_index[0].astype(jnp.int32)
    dst = edge_index[1].astype(jnp.int32)
    pad = jnp.full((EP - E,), N, dtype=jnp.int32)
    srcI = jnp.concatenate([src, pad]).reshape(EP // CHUNK, CHUNK)
    dstI = jnp.concatenate([dst, pad]).reshape(EP // CHUNK, CHUNK)
    x_pad = jnp.pad(x, ((0, NP - N), (0, 0)))

    degp = _sc_degree(dstI)
    xw, y, dinv = _tc_m1(x_pad, W1, degp.T)
    p1 = _sc_agg_rows(y, srcI, dstI)
    z, self2 = _tc_mid(p1, xw, dinv, b1.reshape(1, HID_CH),
                       W2.reshape(1, HID_CH), b2.reshape(1, 1))
    q = _sc_agg_scalar(z.reshape(NP), srcI, dstI)
    out = _tc_fin(q.T, dinv, self2)
    return out[:N, 0]


# speedup vs baseline: 46.9403x; 1.7649x over previous
import functools

import jax
import jax.numpy as jnp
from jax import lax
from jax.experimental import pallas as pl
from jax.experimental.pallas import tpu as pltpu
from jax.experimental.pallas import tpu_sc as plsc

N = 10000
E = 320000
IN_CH = 128
HID_CH = 64

NC = 2
NS = 16
NW = NC * NS
CHUNK = 128
CPT = 80
EPT = CPT * CHUNK
EP = NW * EPT
NP = 10240
RPT = NP // NS
NBUF = 4

_MESH = plsc.VectorSubcoreMesh(
    core_axis_name="c", subcore_axis_name="s", num_cores=NC, num_subcores=NS
)


def _worker(cid, sid):
    return cid * NS + sid


def _zero_1d(ref, n):
    def zrow(i, carry):
        ref[pl.ds(i * 16, 16)] = jnp.zeros((16,), jnp.float32)
        return carry

    lax.fori_loop(0, n // 16, zrow, 0)


@functools.partial(
    pl.kernel,
    out_type=jax.ShapeDtypeStruct((NW, NP), jnp.float32),
    mesh=_MESH,
    compiler_params=pltpu.CompilerParams(needs_layout_passes=False),
    scratch_types=[
        pltpu.VMEM((CPT, CHUNK), jnp.int32),
        pltpu.VMEM((NP,), jnp.float32),
        pltpu.SemaphoreType.DMA,
    ],
)
def _sc_degree(dstI_hbm, out_hbm, didx, accl, isem):
    cid = lax.axis_index("c")
    sid = lax.axis_index("s")
    wid = _worker(cid, sid)
    cp = pltpu.async_copy(dstI_hbm.at[pl.ds(wid * CPT, CPT)], didx, isem)
    _zero_1d(accl, NP)
    cp.wait()
    one = jnp.full((16,), 1.0, jnp.float32)

    def body(r, carry):
        for j in range(CHUNK // 16):
            di = didx[r, pl.ds(j * 16, 16)]
            plsc.addupdate_scatter(accl, [di], one)
        return carry

    lax.fori_loop(0, CPT, body, 0)
    pltpu.sync_copy(accl, out_hbm.at[wid])


CH2 = HID_CH // NC
CPT2 = EP // NS // CHUNK


@functools.partial(
    pl.kernel,
    out_type=jax.ShapeDtypeStruct((NC, NP, CH2), jnp.float32),
    mesh=_MESH,
    compiler_params=pltpu.CompilerParams(use_tc_tiling_on_sc=False),
    scratch_types=[
        pltpu.VMEM((CPT2, CHUNK), jnp.int32),
        pltpu.VMEM((CPT2, CHUNK), jnp.int32),
        [pltpu.VMEM((CHUNK, CH2), jnp.float32)] * NBUF,
        pltpu.VMEM((CHUNK, CH2), jnp.float32),
        pltpu.VMEM_SHARED((NP, CH2), jnp.float32),
        pltpu.VMEM_SHARED((NP, CH2), jnp.float32),
        pltpu.SemaphoreType.DMA,
        [pltpu.SemaphoreType.DMA] * NBUF,
        [pltpu.SemaphoreType.DMA] * NBUF,
    ],
)
def _sc_agg_rows(y2_hbm, srcI_hbm, dstI_hbm, out_hbm, sidx, didx, rows,
                 zbuf, ytab, acc, isem, gsem, ssem):
    cid = lax.axis_index("c")
    sid = lax.axis_index("s")
    cps = pltpu.async_copy(srcI_hbm.at[pl.ds(sid * CPT2, CPT2)], sidx, isem)
    cpd = pltpu.async_copy(dstI_hbm.at[pl.ds(sid * CPT2, CPT2)], didx, isem)
    cpy = pltpu.async_copy(y2_hbm.at[cid, pl.ds(sid * RPT, RPT)],
                           ytab.at[pl.ds(sid * RPT, RPT)], isem)

    def zrow(i, carry):
        for j in range(CH2 // 16):
            zbuf[i, pl.ds(j * 16, 16)] = jnp.zeros((16,), jnp.float32)
        return carry

    lax.fori_loop(0, CHUNK, zrow, 0)
    for k in range(RPT // CHUNK):
        pltpu.sync_copy(zbuf, acc.at[pl.ds(sid * RPT + k * CHUNK, CHUNK)])
    cps.wait()
    cpd.wait()
    cpy.wait()
    plsc.subcore_barrier()

    def body(t, carry):
        c0 = NBUF * t
        gds = []
        for b in range(NBUF):
            @pl.when(t > 0)
            def _drain(b=b):
                pltpu.make_async_copy(rows[b], acc.at[didx.at[0]],
                                      ssem[b]).wait()

            gds.append(
                pltpu.async_copy(ytab.at[sidx.at[c0 + b]], rows[b], gsem[b]))
        for b in range(NBUF):
            gds[b].wait()
            pltpu.async_copy(rows[b], acc.at[didx.at[c0 + b]], ssem[b],
                             add=True)
        return carry

    lax.fori_loop(0, CPT2 // NBUF, body, 0)
    for b in range(NBUF):
        pltpu.make_async_copy(rows[b], acc.at[didx.at[0]], ssem[b]).wait()
    plsc.subcore_barrier()
    pltpu.sync_copy(acc.at[pl.ds(sid * RPT, RPT)],
                    out_hbm.at[cid, pl.ds(sid * RPT, RPT)])


@functools.partial(
    pl.kernel,
    out_type=jax.ShapeDtypeStruct((NW, NP), jnp.float32),
    mesh=_MESH,
    compiler_params=pltpu.CompilerParams(needs_layout_passes=False),
    scratch_types=[
        pltpu.VMEM((CPT, CHUNK), jnp.int32),
        pltpu.VMEM((CPT, CHUNK), jnp.int32),
        pltpu.VMEM((NP,), jnp.float32),
        pltpu.VMEM((NP,), jnp.float32),
        pltpu.SemaphoreType.DMA,
    ],
)
def _sc_agg_scalar(z_hbm, srcI_hbm, dstI_hbm, out_hbm, sidx, didx, zloc,
                   accl, isem):
    cid = lax.axis_index("c")
    sid = lax.axis_index("s")
    wid = _worker(cid, sid)
    cps = pltpu.async_copy(srcI_hbm.at[pl.ds(wid * CPT, CPT)], sidx, isem)
    cpd = pltpu.async_copy(dstI_hbm.at[pl.ds(wid * CPT, CPT)], didx, isem)
    cpz = pltpu.async_copy(z_hbm, zloc, isem)
    _zero_1d(accl, NP)
    cps.wait()
    cpd.wait()
    cpz.wait()

    def body(r, carry):
        for j in range(CHUNK // 16):
            si = sidx[r, pl.ds(j * 16, 16)]
            di = didx[r, pl.ds(j * 16, 16)]
            vals = plsc.load_gather(zloc, [si])
            plsc.addupdate_scatter(accl, [di], vals)
        return carry

    lax.fori_loop(0, CPT, body, 0)
    pltpu.sync_copy(accl, out_hbm.at[wid])


def _tc_m1_body(x_ref, w1_ref, degp_ref, xw_ref, y2_ref, dinv_ref):
    deg = jnp.sum(degp_ref[...], axis=1, keepdims=True) + 1.0
    dinv = lax.rsqrt(deg)
    xw = jnp.dot(x_ref[...], w1_ref[...], preferred_element_type=jnp.float32)
    xw_ref[...] = xw
    y = xw * dinv
    y2_ref[0] = y[:, :CH2]
    y2_ref[1] = y[:, CH2:]
    dinv_ref[...] = dinv


_tc_m1 = pl.pallas_call(
    _tc_m1_body,
    out_shape=[
        jax.ShapeDtypeStruct((NP, HID_CH), jnp.float32),
        jax.ShapeDtypeStruct((NC, NP, CH2), jnp.float32),
        jax.ShapeDtypeStruct((NP, 1), jnp.float32),
    ],
)


def _tc_mid_body(p_ref, xw_ref, dinv_ref, b1_ref, w2t_ref, b2_ref,
                 z_ref, self2_ref):
    dinv = dinv_ref[...]
    agg = jnp.concatenate([p_ref[0], p_ref[1]], axis=1)
    h = jnp.maximum(dinv * agg + (dinv * dinv) * xw_ref[...] + b1_ref[...],
                    0.0)
    hw = jnp.sum(h * w2t_ref[...], axis=1, keepdims=True)
    z_ref[...] = dinv * hw
    self2_ref[...] = (dinv * dinv) * hw + b2_ref[...]


_tc_mid = pl.pallas_call(
    _tc_mid_body,
    out_shape=[
        jax.ShapeDtypeStruct((NP, 1), jnp.float32),
        jax.ShapeDtypeStruct((NP, 1), jnp.float32),
    ],
)


def _tc_fin_body(q_ref, dinv_ref, self2_ref, out_ref):
    q = jnp.sum(q_ref[...], axis=1, keepdims=True)
    out_ref[...] = dinv_ref[...] * q + self2_ref[...]


_tc_fin = pl.pallas_call(
    _tc_fin_body,
    out_shape=jax.ShapeDtypeStruct((NP, 1), jnp.float32),
)


@jax.jit
def kernel(x, edge_index, W1, b1, W2, b2):
    src = edge_index[0].astype(jnp.int32)
    dst = edge_index[1].astype(jnp.int32)
    pad = jnp.full((EP - E,), N, dtype=jnp.int32)
    srcI = jnp.concatenate([src, pad]).reshape(EP // CHUNK, CHUNK)
    dstI = jnp.concatenate([dst, pad]).reshape(EP // CHUNK, CHUNK)
    x_pad = jnp.pad(x, ((0, NP - N), (0, 0)))

    degp = _sc_degree(dstI)
    xw, y, dinv = _tc_m1(x_pad, W1, degp.T)
    p1 = _sc_agg_rows(y, srcI, dstI)
    z, self2 = _tc_mid(p1, xw, dinv, b1.reshape(1, HID_CH),
                       W2.reshape(1, HID_CH), b2.reshape(1, 1))
    q = _sc_agg_scalar(z.reshape(NP), srcI, dstI)
    out = _tc_fin(q.T, dinv, self2)
    return out[:N, 0]


# speedup vs baseline: 49.0661x; 1.0453x over previous
import functools

import jax
import jax.numpy as jnp
from jax import lax
from jax.experimental import pallas as pl
from jax.experimental.pallas import tpu as pltpu
from jax.experimental.pallas import tpu_sc as plsc

N = 10000
E = 320000
IN_CH = 128
HID_CH = 64

NC = 2
NS = 16
NW = NC * NS
CHUNK = 128
EP = 327680
CPT = EP // NS // CHUNK
NP = 10240
RPT = NP // NS
NPH = NP // NC
SPT = NPH // NS
CH2 = HID_CH // NC
NBUF = 8

_MESH = plsc.VectorSubcoreMesh(
    core_axis_name="c", subcore_axis_name="s", num_cores=NC, num_subcores=NS
)


def _fill_1d(ref, n, val):
    v = jnp.full((16,), val, jnp.float32)

    def zrow(i, carry):
        ref[pl.ds(i * 16, 16)] = v
        return carry

    lax.fori_loop(0, n // 16, zrow, 0)


def _stage_indices(idx_hbm, idx_vmem, sid, sem):
    return pltpu.async_copy(idx_hbm.at[pl.ds(sid * CPT, CPT)], idx_vmem, sem)


def _reduce_tiles_via_spmem(accl, shared, rbuf, cid, sid):
    pltpu.sync_copy(accl.at[pl.ds(cid * NPH, NPH)], shared.at[sid])
    plsc.subcore_barrier()
    pltpu.sync_copy(shared.at[pl.ds(0, NS), pl.ds(sid * SPT, SPT)], rbuf)


def _column_sums(rbuf, j):
    s = rbuf[0, pl.ds(j * 16, 16)]
    for t in range(1, NS):
        s = s + rbuf[t, pl.ds(j * 16, 16)]
    return s


@functools.partial(
    pl.kernel,
    out_type=jax.ShapeDtypeStruct((NP,), jnp.float32),
    mesh=_MESH,
    compiler_params=pltpu.CompilerParams(needs_layout_passes=False,
                                         use_tc_tiling_on_sc=False),
    scratch_types=[
        pltpu.VMEM((CPT, CHUNK), jnp.int32),
        pltpu.VMEM((NP,), jnp.float32),
        pltpu.VMEM((NS, SPT), jnp.float32),
        pltpu.VMEM((SPT,), jnp.float32),
        pltpu.VMEM_SHARED((NS, NPH), jnp.float32),
        pltpu.SemaphoreType.DMA,
    ],
)
def _sc_degree(dstI_hbm, out_hbm, didx, accl, rbuf, obuf, shared, isem):
    cid = lax.axis_index("c")
    sid = lax.axis_index("s")
    cp = _stage_indices(dstI_hbm, didx, sid, isem)
    _fill_1d(accl, NP, 1.0)
    cp.wait()
    one = jnp.full((16,), 1.0, jnp.float32)

    def body(r, carry):
        for j in range(CHUNK // 16):
            di = didx[r, pl.ds(j * 16, 16)]
            plsc.addupdate_scatter(accl, [di], one)
        return carry

    lax.fori_loop(0, CPT, body, 0)
    _reduce_tiles_via_spmem(accl, shared, rbuf, cid, sid)
    for j in range(SPT // 16):
        obuf[pl.ds(j * 16, 16)] = _column_sums(rbuf, j) - float(NS - 1)
    pltpu.sync_copy(obuf, out_hbm.at[pl.ds(cid * NPH + sid * SPT, SPT)])


@functools.partial(
    pl.kernel,
    out_type=jax.ShapeDtypeStruct((NC, NP, CH2), jnp.float32),
    mesh=_MESH,
    compiler_params=pltpu.CompilerParams(use_tc_tiling_on_sc=False),
    scratch_types=[
        pltpu.VMEM((CPT, CHUNK), jnp.int32),
        pltpu.VMEM((CPT, CHUNK), jnp.int32),
        [pltpu.VMEM((CHUNK, CH2), jnp.float32)] * NBUF,
        pltpu.VMEM((CHUNK, CH2), jnp.float32),
        pltpu.VMEM_SHARED((NP, CH2), jnp.float32),
        pltpu.VMEM_SHARED((NP, CH2), jnp.float32),
        pltpu.SemaphoreType.DMA,
        [pltpu.SemaphoreType.DMA] * NBUF,
        [pltpu.SemaphoreType.DMA] * NBUF,
    ],
)
def _sc_agg_rows(y2_hbm, srcI_hbm, dstI_hbm, out_hbm, sidx, didx, rows,
                 zbuf, ytab, acc, isem, gsem, ssem):
    cid = lax.axis_index("c")
    sid = lax.axis_index("s")
    cps = _stage_indices(srcI_hbm, sidx, sid, isem)
    cpd = _stage_indices(dstI_hbm, didx, sid, isem)
    cpy = pltpu.async_copy(y2_hbm.at[cid, pl.ds(sid * RPT, RPT)],
                           ytab.at[pl.ds(sid * RPT, RPT)], isem)

    def zrow(i, carry):
        for j in range(CH2 // 16):
            zbuf[i, pl.ds(j * 16, 16)] = jnp.zeros((16,), jnp.float32)
        return carry

    lax.fori_loop(0, CHUNK, zrow, 0)
    for k in range(RPT // CHUNK):
        pltpu.sync_copy(zbuf, acc.at[pl.ds(sid * RPT + k * CHUNK, CHUNK)])
    cps.wait()
    cpd.wait()
    cpy.wait()
    plsc.subcore_barrier()

    def body(t, carry):
        c0 = NBUF * t
        gds = []
        for b in range(NBUF):
            @pl.when(t > 0)
            def _drain(b=b):
                pltpu.make_async_copy(rows[b], acc.at[didx.at[0]],
                                      ssem[b]).wait()

            gds.append(
                pltpu.async_copy(ytab.at[sidx.at[c0 + b]], rows[b], gsem[b]))
        for b in range(NBUF):
            gds[b].wait()
            pltpu.async_copy(rows[b], acc.at[didx.at[c0 + b]], ssem[b],
                             add=True)
        return carry

    lax.fori_loop(0, CPT // NBUF, body, 0)
    for b in range(NBUF):
        pltpu.make_async_copy(rows[b], acc.at[didx.at[0]], ssem[b]).wait()
    plsc.subcore_barrier()
    pltpu.sync_copy(acc.at[pl.ds(sid * RPT, RPT)],
                    out_hbm.at[cid, pl.ds(sid * RPT, RPT)])


@functools.partial(
    pl.kernel,
    out_type=jax.ShapeDtypeStruct((NP,), jnp.float32),
    mesh=_MESH,
    compiler_params=pltpu.CompilerParams(needs_layout_passes=False,
                                         use_tc_tiling_on_sc=False),
    scratch_types=[
        pltpu.VMEM((CPT, CHUNK), jnp.int32),
        pltpu.VMEM((CPT, CHUNK), jnp.int32),
        pltpu.VMEM((NP,), jnp.float32),
        pltpu.VMEM((NP,), jnp.float32),
        pltpu.VMEM((NS, SPT), jnp.float32),
        pltpu.VMEM((SPT,), jnp.float32),
        pltpu.VMEM((SPT,), jnp.float32),
        pltpu.VMEM((SPT,), jnp.float32),
        pltpu.VMEM_SHARED((NS, NPH), jnp.float32),
        pltpu.SemaphoreType.DMA,
    ],
)
def _sc_agg_scalar(z_hbm, srcI_hbm, dstI_hbm, dinv_hbm, self2_hbm, out_hbm,
                   sidx, didx, zloc, accl, rbuf, dbuf, sbuf, obuf, shared,
                   isem):
    cid = lax.axis_index("c")
    sid = lax.axis_index("s")
    off = cid * NPH + sid * SPT
    cps = _stage_indices(srcI_hbm, sidx, sid, isem)
    cpd = _stage_indices(dstI_hbm, didx, sid, isem)
    cpz = pltpu.async_copy(z_hbm, zloc, isem)
    cpdi = pltpu.async_copy(dinv_hbm.at[pl.ds(off, SPT)], dbuf, isem)
    cpse = pltpu.async_copy(self2_hbm.at[pl.ds(off, SPT)], sbuf, isem)
    _fill_1d(accl, NP, 0.0)
    cps.wait()
    cpd.wait()
    cpz.wait()

    def body(r, carry):
        for j in range(CHUNK // 16):
            si = sidx[r, pl.ds(j * 16, 16)]
            di = didx[r, pl.ds(j * 16, 16)]
            vals = plsc.load_gather(zloc, [si])
            plsc.addupdate_scatter(accl, [di], vals)
        return carry

    lax.fori_loop(0, CPT, body, 0)
    _reduce_tiles_via_spmem(accl, shared, rbuf, cid, sid)
    cpdi.wait()
    cpse.wait()
    for j in range(SPT // 16):
        sl = pl.ds(j * 16, 16)
        obuf[sl] = dbuf[sl] * _column_sums(rbuf, j) + sbuf[sl]
    pltpu.sync_copy(obuf, out_hbm.at[pl.ds(off, SPT)])


def _tc_m1_body(x_ref, w1_ref, deg_ref, xw_ref, y2_ref, dinv_ref):
    dinv = lax.rsqrt(deg_ref[...])
    xw = jnp.dot(x_ref[...], w1_ref[...], preferred_element_type=jnp.float32)
    xw_ref[...] = xw
    y = xw * dinv
    y2_ref[0] = y[:, :CH2]
    y2_ref[1] = y[:, CH2:]
    dinv_ref[...] = dinv


_tc_m1 = pl.pallas_call(
    _tc_m1_body,
    out_shape=[
        jax.ShapeDtypeStruct((NP, HID_CH), jnp.float32),
        jax.ShapeDtypeStruct((NC, NP, CH2), jnp.float32),
        jax.ShapeDtypeStruct((NP, 1), jnp.float32),
    ],
)


def _tc_mid_body(p_ref, xw_ref, dinv_ref, b1_ref, w2t_ref, b2_ref,
                 z_ref, self2_ref):
    dinv = dinv_ref[...]
    agg = jnp.concatenate([p_ref[0], p_ref[1]], axis=1)
    h = jnp.maximum(dinv * agg + (dinv * dinv) * xw_ref[...] + b1_ref[...],
                    0.0)
    hw = jnp.sum(h * w2t_ref[...], axis=1, keepdims=True)
    z_ref[...] = dinv * hw
    self2_ref[...] = (dinv * dinv) * hw + b2_ref[...]


_tc_mid = pl.pallas_call(
    _tc_mid_body,
    out_shape=[
        jax.ShapeDtypeStruct((NP, 1), jnp.float32),
        jax.ShapeDtypeStruct((NP, 1), jnp.float32),
    ],
)


@jax.jit
def kernel(x, edge_index, W1, b1, W2, b2):
    src = edge_index[0].astype(jnp.int32)
    dst = edge_index[1].astype(jnp.int32)
    pad = jnp.full((EP - E,), N, dtype=jnp.int32)
    srcI = jnp.concatenate([src, pad]).reshape(EP // CHUNK, CHUNK)
    dstI = jnp.concatenate([dst, pad]).reshape(EP // CHUNK, CHUNK)
    x_pad = jnp.pad(x, ((0, NP - N), (0, 0)))

    deg = _sc_degree(dstI)
    xw, y2, dinv = _tc_m1(x_pad, W1, deg.reshape(NP, 1))
    p1 = _sc_agg_rows(y2, srcI, dstI)
    z, self2 = _tc_mid(p1, xw, dinv, b1.reshape(1, HID_CH),
                       W2.reshape(1, HID_CH), b2.reshape(1, 1))
    out = _sc_agg_scalar(z.reshape(NP), srcI, dstI, dinv.reshape(NP),
                         self2.reshape(NP))
    return out[:N]


# speedup vs baseline: 53.1055x; 1.0823x over previous
import functools

import jax
import jax.numpy as jnp
from jax import lax
from jax.experimental import pallas as pl
from jax.experimental.pallas import tpu as pltpu
from jax.experimental.pallas import tpu_sc as plsc

N = 10000
E = 320000
IN_CH = 128
HID_CH = 64

NC = 2
NS = 16
NW = NC * NS
CHUNK = 128
EP = 327680
CPT = EP // NS // CHUNK
NP = 10240
RPT = NP // NS
NPH = NP // NC
SPT = NPH // NS
CH2 = HID_CH // NC
NBUF = 8

_MESH = plsc.VectorSubcoreMesh(
    core_axis_name="c", subcore_axis_name="s", num_cores=NC, num_subcores=NS
)


def _fill_1d(ref, n, val):
    v = jnp.full((16,), val, jnp.float32)

    def zrow(i, carry):
        ref[pl.ds(i * 16, 16)] = v
        return carry

    lax.fori_loop(0, n // 16, zrow, 0)


def _stage_indices(idx_hbm, idx_vmem, sid, sem):
    return pltpu.async_copy(idx_hbm.at[pl.ds(sid * CPT, CPT)], idx_vmem, sem)


def _reduce_tiles_via_spmem(accl, shared, rbuf, cid, sid):
    pltpu.sync_copy(accl.at[pl.ds(cid * NPH, NPH)], shared.at[sid])
    plsc.subcore_barrier()
    pltpu.sync_copy(shared.at[pl.ds(0, NS), pl.ds(sid * SPT, SPT)], rbuf)


def _column_sums(rbuf, j):
    s = rbuf[0, pl.ds(j * 16, 16)]
    for t in range(1, NS):
        s = s + rbuf[t, pl.ds(j * 16, 16)]
    return s


@functools.partial(
    pl.kernel,
    out_type=jax.ShapeDtypeStruct((NP,), jnp.float32),
    mesh=_MESH,
    compiler_params=pltpu.CompilerParams(needs_layout_passes=False,
                                         use_tc_tiling_on_sc=False),
    scratch_types=[
        pltpu.VMEM((CPT, CHUNK), jnp.int32),
        pltpu.VMEM((NP,), jnp.float32),
        pltpu.VMEM((NS, SPT), jnp.float32),
        pltpu.VMEM((SPT,), jnp.float32),
        pltpu.VMEM_SHARED((NS, NPH), jnp.float32),
        pltpu.SemaphoreType.DMA,
    ],
)
def _sc_degree(dstI_hbm, out_hbm, didx, accl, rbuf, obuf, shared, isem):
    cid = lax.axis_index("c")
    sid = lax.axis_index("s")
    cp = _stage_indices(dstI_hbm, didx, sid, isem)
    _fill_1d(accl, NP, 1.0)
    cp.wait()
    one = jnp.full((16,), 1.0, jnp.float32)

    def body(r, carry):
        for j in range(CHUNK // 16):
            di = didx[r, pl.ds(j * 16, 16)]
            plsc.addupdate_scatter(accl, [di], one)
        return carry

    lax.fori_loop(0, CPT, body, 0)
    _reduce_tiles_via_spmem(accl, shared, rbuf, cid, sid)
    for j in range(SPT // 16):
        obuf[pl.ds(j * 16, 16)] = _column_sums(rbuf, j) - float(NS - 1)
    pltpu.sync_copy(obuf, out_hbm.at[pl.ds(cid * NPH + sid * SPT, SPT)])


@functools.partial(
    pl.kernel,
    out_type=jax.ShapeDtypeStruct((NC, NP, CH2), jnp.float32),
    mesh=_MESH,
    compiler_params=pltpu.CompilerParams(use_tc_tiling_on_sc=False),
    scratch_types=[
        pltpu.VMEM((CPT, CHUNK), jnp.int32),
        pltpu.VMEM((CPT, CHUNK), jnp.int32),
        [pltpu.VMEM((CHUNK, CH2), jnp.float32)] * NBUF,
        pltpu.VMEM((CHUNK, CH2), jnp.float32),
        pltpu.VMEM_SHARED((NP, CH2), jnp.float32),
        pltpu.VMEM_SHARED((NP, CH2), jnp.float32),
        pltpu.SemaphoreType.DMA,
        [pltpu.SemaphoreType.DMA] * NBUF,
        [pltpu.SemaphoreType.DMA] * NBUF,
    ],
)
def _sc_agg_rows(y2_hbm, srcI_hbm, dstI_hbm, out_hbm, sidx, didx, rows,
                 zbuf, ytab, acc, isem, gsem, ssem):
    cid = lax.axis_index("c")
    sid = lax.axis_index("s")
    cps = _stage_indices(srcI_hbm, sidx, sid, isem)
    cpd = _stage_indices(dstI_hbm, didx, sid, isem)
    cpy = pltpu.async_copy(y2_hbm.at[cid, pl.ds(sid * RPT, RPT)],
                           ytab.at[pl.ds(sid * RPT, RPT)], isem)

    def zrow(i, carry):
        for j in range(CH2 // 16):
            zbuf[i, pl.ds(j * 16, 16)] = jnp.zeros((16,), jnp.float32)
        return carry

    lax.fori_loop(0, CHUNK, zrow, 0)
    for k in range(RPT // CHUNK):
        pltpu.sync_copy(zbuf, acc.at[pl.ds(sid * RPT + k * CHUNK, CHUNK)])
    cps.wait()
    cpd.wait()
    cpy.wait()
    plsc.subcore_barrier()

    def body(t, carry):
        c0 = NBUF * t
        gds = []
        for b in range(NBUF):
            @pl.when(t > 0)
            def _drain(b=b):
                pltpu.make_async_copy(rows[b], acc.at[didx.at[0]],
                                      ssem[b]).wait()

            gds.append(
                pltpu.async_copy(ytab.at[sidx.at[c0 + b]], rows[b], gsem[b]))
        for b in range(NBUF):
            gds[b].wait()
            pltpu.async_copy(rows[b], acc.at[didx.at[c0 + b]], ssem[b],
                             add=True)
        return carry

    lax.fori_loop(0, CPT // NBUF, body, 0)
    for b in range(NBUF):
        pltpu.make_async_copy(rows[b], acc.at[didx.at[0]], ssem[b]).wait()
    plsc.subcore_barrier()
    pltpu.sync_copy(acc.at[pl.ds(sid * RPT, RPT)],
                    out_hbm.at[cid, pl.ds(sid * RPT, RPT)])


@functools.partial(
    pl.kernel,
    out_type=jax.ShapeDtypeStruct((NP,), jnp.float32),
    mesh=_MESH,
    compiler_params=pltpu.CompilerParams(needs_layout_passes=False,
                                         use_tc_tiling_on_sc=False),
    scratch_types=[
        pltpu.VMEM((CPT, CHUNK), jnp.int32),
        pltpu.VMEM((CPT, CHUNK), jnp.int32),
        pltpu.VMEM((NP,), jnp.float32),
        pltpu.VMEM((NP,), jnp.float32),
        pltpu.VMEM((NS, SPT), jnp.float32),
        pltpu.VMEM((SPT,), jnp.float32),
        pltpu.VMEM((SPT,), jnp.float32),
        pltpu.VMEM((SPT,), jnp.float32),
        pltpu.VMEM_SHARED((NS, NPH), jnp.float32),
        pltpu.SemaphoreType.DMA,
    ],
)
def _sc_agg_scalar(z_hbm, srcI_hbm, dstI_hbm, dinv_hbm, self2_hbm, out_hbm,
                   sidx, didx, zloc, accl, rbuf, dbuf, sbuf, obuf, shared,
                   isem):
    cid = lax.axis_index("c")
    sid = lax.axis_index("s")
    off = cid * NPH + sid * SPT
    cps = _stage_indices(srcI_hbm, sidx, sid, isem)
    cpd = _stage_indices(dstI_hbm, didx, sid, isem)
    cpz = pltpu.async_copy(z_hbm, zloc, isem)
    cpdi = pltpu.async_copy(dinv_hbm.at[pl.ds(off, SPT)], dbuf, isem)
    cpse = pltpu.async_copy(self2_hbm.at[pl.ds(off, SPT)], sbuf, isem)
    _fill_1d(accl, NP, 0.0)
    cps.wait()
    cpd.wait()
    cpz.wait()

    def body(r, carry):
        for j in range(CHUNK // 16):
            si = sidx[r, pl.ds(j * 16, 16)]
            di = didx[r, pl.ds(j * 16, 16)]
            vals = plsc.load_gather(zloc, [si])
            plsc.addupdate_scatter(accl, [di], vals)
        return carry

    lax.fori_loop(0, CPT, body, 0)
    _reduce_tiles_via_spmem(accl, shared, rbuf, cid, sid)
    cpdi.wait()
    cpse.wait()
    for j in range(SPT // 16):
        sl = pl.ds(j * 16, 16)
        obuf[sl] = dbuf[sl] * _column_sums(rbuf, j) + sbuf[sl]
    pltpu.sync_copy(obuf, out_hbm.at[pl.ds(off, SPT)])


GB = 2048
GP = GB // 128
LANES = 128


def _lane_diag(r):
    li = lax.broadcasted_iota(jnp.int32, (r, LANES), 0) % LANES
    ci = lax.broadcasted_iota(jnp.int32, (r, LANES), 1)
    return (li == ci).astype(jnp.float32)


def _col_from_planar(p):
    gp, _ = p.shape
    r = gp * LANES
    ri = lax.broadcasted_iota(jnp.int32, (r, gp), 0)
    ji = lax.broadcasted_iota(jnp.int32, (r, gp), 1)
    a = (ri // LANES == ji).astype(jnp.float32)
    expanded = jnp.dot(a, p, preferred_element_type=jnp.float32)
    return jnp.sum(expanded * _lane_diag(r), axis=1, keepdims=True)


def _planar_from_col(c, gp):
    r = c.shape[0]
    m = c * _lane_diag(r)
    ji = lax.broadcasted_iota(jnp.int32, (gp, r), 0)
    ri = lax.broadcasted_iota(jnp.int32, (gp, r), 1)
    at = (ri // LANES == ji).astype(jnp.float32)
    return jnp.dot(at, m, preferred_element_type=jnp.float32)


def _tc_m1_body(x_ref, w1_ref, deg_ref, xw_ref, y2_ref, dinv_ref):
    dinvp = lax.rsqrt(deg_ref[...])
    dinv_ref[...] = dinvp
    dcol = _col_from_planar(dinvp)
    xw = jnp.dot(x_ref[...], w1_ref[...], preferred_element_type=jnp.float32)
    xw_ref[...] = xw
    y = xw * dcol
    y2_ref[0] = y[:, :CH2]
    y2_ref[1] = y[:, CH2:]


_tc_m1 = pl.pallas_call(
    _tc_m1_body,
    grid=(NP // GB,),
    in_specs=[
        pl.BlockSpec((GB, IN_CH), lambda i: (i, 0)),
        pl.BlockSpec((IN_CH, HID_CH), lambda i: (0, 0)),
        pl.BlockSpec((GP, LANES), lambda i: (i, 0)),
    ],
    out_specs=[
        pl.BlockSpec((GB, HID_CH), lambda i: (i, 0)),
        pl.BlockSpec((NC, GB, CH2), lambda i: (0, i, 0)),
        pl.BlockSpec((GP, LANES), lambda i: (i, 0)),
    ],
    out_shape=[
        jax.ShapeDtypeStruct((NP, HID_CH), jnp.float32),
        jax.ShapeDtypeStruct((NC, NP, CH2), jnp.float32),
        jax.ShapeDtypeStruct((NP // LANES, LANES), jnp.float32),
    ],
)


def _tc_mid_body(p_ref, xw_ref, dinv_ref, b1_ref, w2t_ref, b2_ref,
                 z_ref, self2_ref):
    dinvp = dinv_ref[...]
    dcol = _col_from_planar(dinvp)
    agg = jnp.concatenate([p_ref[0], p_ref[1]], axis=1)
    h = jnp.maximum(dcol * agg + (dcol * dcol) * xw_ref[...] + b1_ref[...],
                    0.0)
    hw = jnp.sum(h * w2t_ref[...], axis=1, keepdims=True)
    hwp = _planar_from_col(hw, GP)
    z_ref[...] = dinvp * hwp
    self2_ref[...] = dinvp * dinvp * hwp + b2_ref[...]


_tc_mid = pl.pallas_call(
    _tc_mid_body,
    grid=(NP // GB,),
    in_specs=[
        pl.BlockSpec((NC, GB, CH2), lambda i: (0, i, 0)),
        pl.BlockSpec((GB, HID_CH), lambda i: (i, 0)),
        pl.BlockSpec((GP, LANES), lambda i: (i, 0)),
        pl.BlockSpec((1, HID_CH), lambda i: (0, 0)),
        pl.BlockSpec((1, HID_CH), lambda i: (0, 0)),
        pl.BlockSpec((1, 1), lambda i: (0, 0)),
    ],
    out_specs=[
        pl.BlockSpec((GP, LANES), lambda i: (i, 0)),
        pl.BlockSpec((GP, LANES), lambda i: (i, 0)),
    ],
    out_shape=[
        jax.ShapeDtypeStruct((NP // LANES, LANES), jnp.float32),
        jax.ShapeDtypeStruct((NP // LANES, LANES), jnp.float32),
    ],
)


@jax.jit
def kernel(x, edge_index, W1, b1, W2, b2):
    src = edge_index[0].astype(jnp.int32)
    dst = edge_index[1].astype(jnp.int32)
    pad = jnp.full((EP - E,), N, dtype=jnp.int32)
    srcI = jnp.concatenate([src, pad]).reshape(EP // CHUNK, CHUNK)
    dstI = jnp.concatenate([dst, pad]).reshape(EP // CHUNK, CHUNK)
    x_pad = jnp.pad(x, ((0, NP - N), (0, 0)))

    deg = _sc_degree(dstI)
    xw, y2, dinvp = _tc_m1(x_pad, W1, deg.reshape(NP // LANES, LANES))
    p1 = _sc_agg_rows(y2, srcI, dstI)
    z, self2 = _tc_mid(p1, xw, dinvp, b1.reshape(1, HID_CH),
                       W2.reshape(1, HID_CH), b2.reshape(1, 1))
    out = _sc_agg_scalar(z.reshape(NP), srcI, dstI, dinvp.reshape(NP),
                         self2.reshape(NP))
    return out[:N]


# speedup vs baseline: 56.5366x; 1.0646x over previous
import functools

import jax
import jax.numpy as jnp
from jax import lax
from jax.experimental import pallas as pl
from jax.experimental.pallas import tpu as pltpu
from jax.experimental.pallas import tpu_sc as plsc

N = 10000
E = 320000
IN_CH = 128
HID_CH = 64

NC = 2
NS = 16
NW = NC * NS
CHUNK = 128
EP = 327680
CPT = EP // NS // CHUNK
NP = 10240
RPT = NP // NS
NPH = NP // NC
SPT = NPH // NS
CH2 = HID_CH // NC
NBUF = 8

_MESH = plsc.VectorSubcoreMesh(
    core_axis_name="c", subcore_axis_name="s", num_cores=NC, num_subcores=NS
)


def _fill_1d(ref, n, val):
    v = jnp.full((16,), val, jnp.float32)

    def zrow(i, carry):
        ref[pl.ds(i * 16, 16)] = v
        return carry

    lax.fori_loop(0, n // 16, zrow, 0)


def _stage_indices(idx_hbm, idx_vmem, sid, sem):
    return pltpu.async_copy(idx_hbm.at[pl.ds(sid * CPT, CPT)], idx_vmem, sem)


def _reduce_tiles_via_spmem(accl, shared, rbuf, cid, sid):
    pltpu.sync_copy(accl.at[pl.ds(cid * NPH, NPH)], shared.at[sid])
    plsc.subcore_barrier()
    pltpu.sync_copy(shared.at[pl.ds(0, NS), pl.ds(sid * SPT, SPT)], rbuf)


def _column_sums(rbuf, j):
    s = rbuf[0, pl.ds(j * 16, 16)]
    for t in range(1, NS):
        s = s + rbuf[t, pl.ds(j * 16, 16)]
    return s


@functools.partial(
    pl.kernel,
    out_type=jax.ShapeDtypeStruct((NP,), jnp.float32),
    mesh=_MESH,
    compiler_params=pltpu.CompilerParams(needs_layout_passes=False,
                                         use_tc_tiling_on_sc=False),
    scratch_types=[
        pltpu.VMEM((CPT, CHUNK), jnp.int32),
        pltpu.VMEM((NP,), jnp.float32),
        pltpu.VMEM((NS, SPT), jnp.float32),
        pltpu.VMEM((SPT,), jnp.float32),
        pltpu.VMEM_SHARED((NS, NPH), jnp.float32),
        pltpu.SemaphoreType.DMA,
    ],
)
def _sc_degree(dstI_hbm, out_hbm, didx, accl, rbuf, obuf, shared, isem):
    cid = lax.axis_index("c")
    sid = lax.axis_index("s")
    cp = _stage_indices(dstI_hbm, didx, sid, isem)
    _fill_1d(accl, NP, 1.0)
    cp.wait()
    one = jnp.full((16,), 1.0, jnp.float32)

    def body(r, carry):
        for j in range(CHUNK // 16):
            di = didx[r, pl.ds(j * 16, 16)]
            plsc.addupdate_scatter(accl, [di], one)
        return carry

    lax.fori_loop(0, CPT, body, 0)
    _reduce_tiles_via_spmem(accl, shared, rbuf, cid, sid)
    for j in range(SPT // 16):
        obuf[pl.ds(j * 16, 16)] = _column_sums(rbuf, j) - float(NS - 1)
    pltpu.sync_copy(obuf, out_hbm.at[pl.ds(cid * NPH + sid * SPT, SPT)])


@functools.partial(
    pl.kernel,
    out_type=jax.ShapeDtypeStruct((NC, NP, 128), jnp.float32),
    mesh=_MESH,
    compiler_params=pltpu.CompilerParams(use_tc_tiling_on_sc=False),
    scratch_types=[
        pltpu.VMEM((CPT, CHUNK), jnp.int32),
        pltpu.VMEM((CPT, CHUNK), jnp.int32),
        [pltpu.VMEM((CHUNK, CH2), jnp.float32)] * NBUF,
        pltpu.VMEM((CHUNK, CH2), jnp.float32),
        pltpu.VMEM_SHARED((NP, CH2), jnp.float32),
        pltpu.VMEM_SHARED((NP, CH2), jnp.float32),
        pltpu.SemaphoreType.DMA,
        [pltpu.SemaphoreType.DMA] * NBUF,
        [pltpu.SemaphoreType.DMA] * NBUF,
    ],
)
def _sc_agg_rows(y2_hbm, srcI_hbm, dstI_hbm, out_hbm, sidx, didx, rows,
                 zbuf, ytab, acc, isem, gsem, ssem):
    cid = lax.axis_index("c")
    sid = lax.axis_index("s")
    cps = _stage_indices(srcI_hbm, sidx, sid, isem)
    cpd = _stage_indices(dstI_hbm, didx, sid, isem)
    cpy = pltpu.async_copy(
        y2_hbm.at[cid, pl.ds(sid * RPT, RPT), pl.ds(0, CH2)],
        ytab.at[pl.ds(sid * RPT, RPT)], isem)

    def zrow(i, carry):
        for j in range(CH2 // 16):
            zbuf[i, pl.ds(j * 16, 16)] = jnp.zeros((16,), jnp.float32)
        return carry

    lax.fori_loop(0, CHUNK, zrow, 0)
    for k in range(RPT // CHUNK):
        pltpu.sync_copy(zbuf, acc.at[pl.ds(sid * RPT + k * CHUNK, CHUNK)])
    cps.wait()
    cpd.wait()
    cpy.wait()
    plsc.subcore_barrier()

    def body(t, carry):
        c0 = NBUF * t
        gds = []
        for b in range(NBUF):
            @pl.when(t > 0)
            def _drain(b=b):
                pltpu.make_async_copy(rows[b], acc.at[didx.at[0]],
                                      ssem[b]).wait()

            gds.append(
                pltpu.async_copy(ytab.at[sidx.at[c0 + b]], rows[b], gsem[b]))
        for b in range(NBUF):
            gds[b].wait()
            pltpu.async_copy(rows[b], acc.at[didx.at[c0 + b]], ssem[b],
                             add=True)
        return carry

    lax.fori_loop(0, CPT // NBUF, body, 0)
    for b in range(NBUF):
        pltpu.make_async_copy(rows[b], acc.at[didx.at[0]], ssem[b]).wait()
    plsc.subcore_barrier()
    pltpu.sync_copy(acc.at[pl.ds(sid * RPT, RPT)],
                    out_hbm.at[cid, pl.ds(sid * RPT, RPT), pl.ds(0, CH2)])


@functools.partial(
    pl.kernel,
    out_type=jax.ShapeDtypeStruct((NP,), jnp.float32),
    mesh=_MESH,
    compiler_params=pltpu.CompilerParams(needs_layout_passes=False,
                                         use_tc_tiling_on_sc=False),
    scratch_types=[
        pltpu.VMEM((CPT, CHUNK), jnp.int32),
        pltpu.VMEM((CPT, CHUNK), jnp.int32),
        pltpu.VMEM((NP,), jnp.float32),
        pltpu.VMEM((NP,), jnp.float32),
        pltpu.VMEM((NS, SPT), jnp.float32),
        pltpu.VMEM((SPT,), jnp.float32),
        pltpu.VMEM((SPT,), jnp.float32),
        pltpu.VMEM((SPT,), jnp.float32),
        pltpu.VMEM_SHARED((NS, NPH), jnp.float32),
        pltpu.SemaphoreType.DMA,
    ],
)
def _sc_agg_scalar(z_hbm, srcI_hbm, dstI_hbm, dinv_hbm, self2_hbm, out_hbm,
                   sidx, didx, zloc, accl, rbuf, dbuf, sbuf, obuf, shared,
                   isem):
    cid = lax.axis_index("c")
    sid = lax.axis_index("s")
    off = cid * NPH + sid * SPT
    cps = _stage_indices(srcI_hbm, sidx, sid, isem)
    cpd = _stage_indices(dstI_hbm, didx, sid, isem)
    cpz = pltpu.async_copy(z_hbm, zloc, isem)
    cpdi = pltpu.async_copy(dinv_hbm.at[pl.ds(off, SPT)], dbuf, isem)
    cpse = pltpu.async_copy(self2_hbm.at[pl.ds(off, SPT)], sbuf, isem)
    _fill_1d(accl, NP, 0.0)
    cps.wait()
    cpd.wait()
    cpz.wait()

    def body(r, carry):
        for j in range(CHUNK // 16):
            si = sidx[r, pl.ds(j * 16, 16)]
            di = didx[r, pl.ds(j * 16, 16)]
            vals = plsc.load_gather(zloc, [si])
            plsc.addupdate_scatter(accl, [di], vals)
        return carry

    lax.fori_loop(0, CPT, body, 0)
    _reduce_tiles_via_spmem(accl, shared, rbuf, cid, sid)
    cpdi.wait()
    cpse.wait()
    for j in range(SPT // 16):
        sl = pl.ds(j * 16, 16)
        obuf[sl] = dbuf[sl] * _column_sums(rbuf, j) + sbuf[sl]
    pltpu.sync_copy(obuf, out_hbm.at[pl.ds(off, SPT)])


GB = 2048
GP = GB // 128
LANES = 128


def _lane_diag(r):
    li = lax.broadcasted_iota(jnp.int32, (r, LANES), 0) % LANES
    ci = lax.broadcasted_iota(jnp.int32, (r, LANES), 1)
    return (li == ci).astype(jnp.float32)


def _col_from_planar(p):
    gp, _ = p.shape
    r = gp * LANES
    ri = lax.broadcasted_iota(jnp.int32, (r, gp), 0)
    ji = lax.broadcasted_iota(jnp.int32, (r, gp), 1)
    a = (ri // LANES == ji).astype(jnp.float32)
    expanded = jnp.dot(a, p, preferred_element_type=jnp.float32)
    return jnp.sum(expanded * _lane_diag(r), axis=1, keepdims=True)


def _planar_from_col(c, gp):
    r = c.shape[0]
    m = c * _lane_diag(r)
    ji = lax.broadcasted_iota(jnp.int32, (gp, r), 0)
    ri = lax.broadcasted_iota(jnp.int32, (gp, r), 1)
    at = (ri // LANES == ji).astype(jnp.float32)
    return jnp.dot(at, m, preferred_element_type=jnp.float32)


def _tc_m1_body(x_ref, w1_ref, deg_ref, xw_ref, y2_ref, dinv_ref):
    dinvp = lax.rsqrt(deg_ref[...])
    dinv_ref[...] = dinvp
    dcol = _col_from_planar(dinvp)
    xw = jnp.dot(x_ref[...], w1_ref[...], preferred_element_type=jnp.float32)
    xw_ref[...] = xw
    y = xw * dcol
    pad = jnp.zeros((y.shape[0], 128 - HID_CH), jnp.float32)
    y2_ref[0] = jnp.concatenate([y[:, :CH2], y[:, CH2:], pad], axis=1)
    y2_ref[1] = jnp.concatenate([y[:, CH2:], y[:, :CH2], pad], axis=1)


_tc_m1 = pl.pallas_call(
    _tc_m1_body,
    grid=(NP // GB,),
    in_specs=[
        pl.BlockSpec((GB, IN_CH), lambda i: (i, 0)),
        pl.BlockSpec((IN_CH, HID_CH), lambda i: (0, 0)),
        pl.BlockSpec((GP, LANES), lambda i: (i, 0)),
    ],
    out_specs=[
        pl.BlockSpec((GB, HID_CH), lambda i: (i, 0)),
        pl.BlockSpec((NC, GB, LANES), lambda i: (0, i, 0)),
        pl.BlockSpec((GP, LANES), lambda i: (i, 0)),
    ],
    out_shape=[
        jax.ShapeDtypeStruct((NP, HID_CH), jnp.float32),
        jax.ShapeDtypeStruct((NC, NP, LANES), jnp.float32),
        jax.ShapeDtypeStruct((NP // LANES, LANES), jnp.float32),
    ],
)


def _tc_mid_body(p_ref, xw_ref, dinv_ref, b1_ref, w2t_ref, b2_ref,
                 z_ref, self2_ref):
    dinvp = dinv_ref[...]
    dcol = _col_from_planar(dinvp)
    agg = jnp.concatenate([p_ref[0][:, :CH2], p_ref[1][:, :CH2]],
                          axis=1)
    h = jnp.maximum(dcol * agg + (dcol * dcol) * xw_ref[...] + b1_ref[...],
                    0.0)
    hw = jnp.sum(h * w2t_ref[...], axis=1, keepdims=True)
    hwp = _planar_from_col(hw, GP)
    z_ref[...] = dinvp * hwp
    self2_ref[...] = dinvp * dinvp * hwp + b2_ref[...]


_tc_mid = pl.pallas_call(
    _tc_mid_body,
    grid=(NP // GB,),
    in_specs=[
        pl.BlockSpec((NC, GB, LANES), lambda i: (0, i, 0)),
        pl.BlockSpec((GB, HID_CH), lambda i: (i, 0)),
        pl.BlockSpec((GP, LANES), lambda i: (i, 0)),
        pl.BlockSpec((1, HID_CH), lambda i: (0, 0)),
        pl.BlockSpec((1, HID_CH), lambda i: (0, 0)),
        pl.BlockSpec((1, 1), lambda i: (0, 0)),
    ],
    out_specs=[
        pl.BlockSpec((GP, LANES), lambda i: (i, 0)),
        pl.BlockSpec((GP, LANES), lambda i: (i, 0)),
    ],
    out_shape=[
        jax.ShapeDtypeStruct((NP // LANES, LANES), jnp.float32),
        jax.ShapeDtypeStruct((NP // LANES, LANES), jnp.float32),
    ],
)


@jax.jit
def kernel(x, edge_index, W1, b1, W2, b2):
    src = edge_index[0].astype(jnp.int32)
    dst = edge_index[1].astype(jnp.int32)
    pad = jnp.full((EP - E,), N, dtype=jnp.int32)
    srcI = jnp.concatenate([src, pad]).reshape(EP // CHUNK, CHUNK)
    dstI = jnp.concatenate([dst, pad]).reshape(EP // CHUNK, CHUNK)
    x_pad = jnp.pad(x, ((0, NP - N), (0, 0)))

    deg = _sc_degree(dstI)
    xw, y2, dinvp = _tc_m1(x_pad, W1, deg.reshape(NP // LANES, LANES))
    p1 = _sc_agg_rows(y2, srcI, dstI)
    z, self2 = _tc_mid(p1, xw, dinvp, b1.reshape(1, HID_CH),
                       W2.reshape(1, HID_CH), b2.reshape(1, 1))
    out = _sc_agg_scalar(z.reshape(NP), srcI, dstI, dinvp.reshape(NP),
                         self2.reshape(NP))
    return out[:N]


# speedup vs baseline: 59.5706x; 1.0537x over previous
import functools

import jax
import jax.numpy as jnp
from jax import lax
from jax.experimental import pallas as pl
from jax.experimental.pallas import tpu as pltpu
from jax.experimental.pallas import tpu_sc as plsc

N = 10000
E = 320000
IN_CH = 128
HID_CH = 64

NC = 2
NS = 16
NW = NC * NS
CHUNK = 128
EP = 327680
CPT = EP // NS // CHUNK
NP = 10240
RPT = NP // NS
NPH = NP // NC
SPT = NPH // NS
CH2 = HID_CH // NC
NBUF = 8

_MESH = plsc.VectorSubcoreMesh(
    core_axis_name="c", subcore_axis_name="s", num_cores=NC, num_subcores=NS
)


def _fill_1d(ref, n, val):
    v = jnp.full((16,), val, jnp.float32)

    def zrow(i, carry):
        ref[pl.ds(i * 16, 16)] = v
        return carry

    lax.fori_loop(0, n // 16, zrow, 0)


def _stage_indices(idx_hbm, idx_vmem, sid, sem):
    return pltpu.async_copy(idx_hbm.at[pl.ds(sid * CPT, CPT)], idx_vmem, sem)


def _reduce_tiles_via_spmem(accl, shared, rbuf, cid, sid):
    pltpu.sync_copy(accl.at[pl.ds(cid * NPH, NPH)], shared.at[sid])
    plsc.subcore_barrier()
    pltpu.sync_copy(shared.at[pl.ds(0, NS), pl.ds(sid * SPT, SPT)], rbuf)


def _column_sums(rbuf, j):
    s = rbuf[0, pl.ds(j * 16, 16)]
    for t in range(1, NS):
        s = s + rbuf[t, pl.ds(j * 16, 16)]
    return s


@functools.partial(
    pl.kernel,
    out_type=jax.ShapeDtypeStruct((NP,), jnp.float32),
    mesh=_MESH,
    compiler_params=pltpu.CompilerParams(needs_layout_passes=False,
                                         use_tc_tiling_on_sc=False),
    scratch_types=[
        pltpu.VMEM((CPT, CHUNK), jnp.int32),
        pltpu.VMEM((NP,), jnp.float32),
        pltpu.VMEM((NS, SPT), jnp.float32),
        pltpu.VMEM((SPT,), jnp.float32),
        pltpu.VMEM_SHARED((NS, NPH), jnp.float32),
        pltpu.SemaphoreType.DMA,
    ],
)
def _sc_degree(dstI_hbm, out_hbm, didx, accl, rbuf, obuf, shared, isem):
    cid = lax.axis_index("c")
    sid = lax.axis_index("s")
    cp = _stage_indices(dstI_hbm, didx, sid, isem)
    _fill_1d(accl, NP, 1.0)
    cp.wait()
    one = jnp.full((16,), 1.0, jnp.float32)

    def body(r, carry):
        for j in range(CHUNK // 16):
            di = didx[r, pl.ds(j * 16, 16)]
            plsc.addupdate_scatter(accl, [di], one)
        return carry

    lax.fori_loop(0, CPT, body, 0)
    _reduce_tiles_via_spmem(accl, shared, rbuf, cid, sid)
    for j in range(SPT // 16):
        obuf[pl.ds(j * 16, 16)] = _column_sums(rbuf, j) - float(NS - 1)
    pltpu.sync_copy(obuf, out_hbm.at[pl.ds(cid * NPH + sid * SPT, SPT)])


@functools.partial(
    pl.kernel,
    out_type=jax.ShapeDtypeStruct((NP, 128), jnp.float32),
    mesh=_MESH,
    compiler_params=pltpu.CompilerParams(use_tc_tiling_on_sc=False),
    scratch_types=[
        pltpu.VMEM((CPT, CHUNK), jnp.int32),
        pltpu.VMEM((CPT, CHUNK), jnp.int32),
        [pltpu.VMEM((CHUNK, CH2), jnp.float32)] * NBUF,
        pltpu.VMEM((CHUNK, CH2), jnp.float32),
        pltpu.VMEM_SHARED((NP, CH2), jnp.float32),
        pltpu.VMEM_SHARED((NP, CH2), jnp.float32),
        pltpu.SemaphoreType.DMA,
        [pltpu.SemaphoreType.DMA] * NBUF,
        [pltpu.SemaphoreType.DMA] * NBUF,
    ],
)
def _sc_agg_rows(y2_hbm, srcI_hbm, dstI_hbm, out_hbm, sidx, didx, rows,
                 zbuf, ytab, acc, isem, gsem, ssem):
    cid = lax.axis_index("c")
    sid = lax.axis_index("s")
    cps = _stage_indices(srcI_hbm, sidx, sid, isem)
    cpd = _stage_indices(dstI_hbm, didx, sid, isem)
    cpy = pltpu.async_copy(
        y2_hbm.at[pl.ds(sid * RPT, RPT), pl.ds(cid * CH2, CH2)],
        ytab.at[pl.ds(sid * RPT, RPT)], isem)

    def zrow(i, carry):
        for j in range(CH2 // 16):
            zbuf[i, pl.ds(j * 16, 16)] = jnp.zeros((16,), jnp.float32)
        return carry

    lax.fori_loop(0, CHUNK, zrow, 0)
    for k in range(RPT // CHUNK):
        pltpu.sync_copy(zbuf, acc.at[pl.ds(sid * RPT + k * CHUNK, CHUNK)])
    cps.wait()
    cpd.wait()
    cpy.wait()
    plsc.subcore_barrier()

    def body(t, carry):
        c0 = NBUF * t
        gds = []
        for b in range(NBUF):
            @pl.when(t > 0)
            def _drain(b=b):
                pltpu.make_async_copy(rows[b], acc.at[didx.at[0]],
                                      ssem[b]).wait()

            gds.append(
                pltpu.async_copy(ytab.at[sidx.at[c0 + b]], rows[b], gsem[b]))
        for b in range(NBUF):
            gds[b].wait()
            pltpu.async_copy(rows[b], acc.at[didx.at[c0 + b]], ssem[b],
                             add=True)
        return carry

    lax.fori_loop(0, CPT // NBUF, body, 0)
    for b in range(NBUF):
        pltpu.make_async_copy(rows[b], acc.at[didx.at[0]], ssem[b]).wait()
    plsc.subcore_barrier()
    pltpu.sync_copy(acc.at[pl.ds(sid * RPT, RPT)],
                    out_hbm.at[pl.ds(sid * RPT, RPT), pl.ds(cid * CH2, CH2)])


@functools.partial(
    pl.kernel,
    out_type=jax.ShapeDtypeStruct((NP,), jnp.float32),
    mesh=_MESH,
    compiler_params=pltpu.CompilerParams(needs_layout_passes=False,
                                         use_tc_tiling_on_sc=False),
    scratch_types=[
        pltpu.VMEM((CPT, CHUNK), jnp.int32),
        pltpu.VMEM((CPT, CHUNK), jnp.int32),
        pltpu.VMEM((NP,), jnp.float32),
        pltpu.VMEM((NP,), jnp.float32),
        pltpu.VMEM((NS, SPT), jnp.float32),
        pltpu.VMEM((SPT,), jnp.float32),
        pltpu.VMEM((SPT,), jnp.float32),
        pltpu.VMEM((SPT,), jnp.float32),
        pltpu.VMEM_SHARED((NS, NPH), jnp.float32),
        pltpu.SemaphoreType.DMA,
    ],
)
def _sc_agg_scalar(z_hbm, srcI_hbm, dstI_hbm, dinv_hbm, self2_hbm, out_hbm,
                   sidx, didx, zloc, accl, rbuf, dbuf, sbuf, obuf, shared,
                   isem):
    cid = lax.axis_index("c")
    sid = lax.axis_index("s")
    off = cid * NPH + sid * SPT
    cps = _stage_indices(srcI_hbm, sidx, sid, isem)
    cpd = _stage_indices(dstI_hbm, didx, sid, isem)
    cpz = pltpu.async_copy(z_hbm, zloc, isem)
    cpdi = pltpu.async_copy(dinv_hbm.at[pl.ds(off, SPT)], dbuf, isem)
    cpse = pltpu.async_copy(self2_hbm.at[pl.ds(off, SPT)], sbuf, isem)
    _fill_1d(accl, NP, 0.0)
    cps.wait()
    cpd.wait()
    cpz.wait()

    def body(r, carry):
        for j in range(CHUNK // 16):
            si = sidx[r, pl.ds(j * 16, 16)]
            di = didx[r, pl.ds(j * 16, 16)]
            vals = plsc.load_gather(zloc, [si])
            plsc.addupdate_scatter(accl, [di], vals)
        return carry

    lax.fori_loop(0, CPT, body, 0)
    _reduce_tiles_via_spmem(accl, shared, rbuf, cid, sid)
    cpdi.wait()
    cpse.wait()
    for j in range(SPT // 16):
        sl = pl.ds(j * 16, 16)
        obuf[sl] = dbuf[sl] * _column_sums(rbuf, j) + sbuf[sl]
    pltpu.sync_copy(obuf, out_hbm.at[pl.ds(off, SPT)])


GB = 2048
GP = GB // 128
LANES = 128


def _lane_diag(r):
    li = lax.broadcasted_iota(jnp.int32, (r, LANES), 0) % LANES
    ci = lax.broadcasted_iota(jnp.int32, (r, LANES), 1)
    return (li == ci).astype(jnp.float32)


def _col_from_planar(p):
    gp, _ = p.shape
    r = gp * LANES
    ri = lax.broadcasted_iota(jnp.int32, (r, gp), 0)
    ji = lax.broadcasted_iota(jnp.int32, (r, gp), 1)
    a = (ri // LANES == ji).astype(jnp.float32)
    expanded = jnp.dot(a, p, preferred_element_type=jnp.float32)
    return jnp.sum(expanded * _lane_diag(r), axis=1, keepdims=True)


def _planar_from_col(c, gp):
    r = c.shape[0]
    m = c * _lane_diag(r)
    ji = lax.broadcasted_iota(jnp.int32, (gp, r), 0)
    ri = lax.broadcasted_iota(jnp.int32, (gp, r), 1)
    at = (ri // LANES == ji).astype(jnp.float32)
    return jnp.dot(at, m, preferred_element_type=jnp.float32)


def _tc_mm_body(x_ref, w1_ref, xw_ref):
    xw_ref[...] = jnp.dot(x_ref[...], w1_ref[...],
                          preferred_element_type=jnp.float32)


_tc_mm = pl.pallas_call(
    _tc_mm_body,
    grid=(NP // GB,),
    in_specs=[
        pl.BlockSpec((GB, IN_CH), lambda i: (i, 0)),
        pl.BlockSpec((IN_CH, HID_CH), lambda i: (0, 0)),
    ],
    out_specs=pl.BlockSpec((GB, HID_CH), lambda i: (i, 0)),
    out_shape=jax.ShapeDtypeStruct((NP, HID_CH), jnp.float32),
)


def _tc_scale_body(xw_ref, deg_ref, y2_ref, dinv_ref):
    dinvp = lax.rsqrt(deg_ref[...])
    dinv_ref[...] = dinvp
    dcol = _col_from_planar(dinvp)
    y = xw_ref[...] * dcol
    pad = jnp.zeros((y.shape[0], 128 - HID_CH), jnp.float32)
    y2_ref[...] = jnp.concatenate([y, pad], axis=1)


_tc_scale = pl.pallas_call(
    _tc_scale_body,
    grid=(NP // GB,),
    in_specs=[
        pl.BlockSpec((GB, HID_CH), lambda i: (i, 0)),
        pl.BlockSpec((GP, LANES), lambda i: (i, 0)),
    ],
    out_specs=[
        pl.BlockSpec((GB, LANES), lambda i: (i, 0)),
        pl.BlockSpec((GP, LANES), lambda i: (i, 0)),
    ],
    out_shape=[
        jax.ShapeDtypeStruct((NP, LANES), jnp.float32),
        jax.ShapeDtypeStruct((NP // LANES, LANES), jnp.float32),
    ],
)


def _tc_mid_body(p_ref, xw_ref, dinv_ref, b1_ref, w2t_ref, b2_ref,
                 z_ref, self2_ref):
    dinvp = dinv_ref[...]
    dcol = _col_from_planar(dinvp)
    agg = p_ref[:, :HID_CH]
    h = jnp.maximum(dcol * agg + (dcol * dcol) * xw_ref[...] + b1_ref[...],
                    0.0)
    hw = jnp.sum(h * w2t_ref[...], axis=1, keepdims=True)
    hwp = _planar_from_col(hw, GP)
    z_ref[...] = dinvp * hwp
    self2_ref[...] = dinvp * dinvp * hwp + b2_ref[...]


_tc_mid = pl.pallas_call(
    _tc_mid_body,
    grid=(NP // GB,),
    in_specs=[
        pl.BlockSpec((GB, LANES), lambda i: (i, 0)),
        pl.BlockSpec((GB, HID_CH), lambda i: (i, 0)),
        pl.BlockSpec((GP, LANES), lambda i: (i, 0)),
        pl.BlockSpec((1, HID_CH), lambda i: (0, 0)),
        pl.BlockSpec((1, HID_CH), lambda i: (0, 0)),
        pl.BlockSpec((1, 1), lambda i: (0, 0)),
    ],
    out_specs=[
        pl.BlockSpec((GP, LANES), lambda i: (i, 0)),
        pl.BlockSpec((GP, LANES), lambda i: (i, 0)),
    ],
    out_shape=[
        jax.ShapeDtypeStruct((NP // LANES, LANES), jnp.float32),
        jax.ShapeDtypeStruct((NP // LANES, LANES), jnp.float32),
    ],
)


@jax.jit
def kernel(x, edge_index, W1, b1, W2, b2):
    src = edge_index[0].astype(jnp.int32)
    dst = edge_index[1].astype(jnp.int32)
    pad = jnp.full((EP - E,), N, dtype=jnp.int32)
    srcI = jnp.concatenate([src, pad]).reshape(EP // CHUNK, CHUNK)
    dstI = jnp.concatenate([dst, pad]).reshape(EP // CHUNK, CHUNK)
    x_pad = jnp.pad(x, ((0, NP - N), (0, 0)))

    deg = _sc_degree(dstI)
    xw = _tc_mm(x_pad, W1)
    y2, dinvp = _tc_scale(xw, deg.reshape(NP // LANES, LANES))
    p1 = _sc_agg_rows(y2, srcI, dstI)
    z, self2 = _tc_mid(p1, xw, dinvp, b1.reshape(1, HID_CH),
                       W2.reshape(1, HID_CH), b2.reshape(1, 1))
    out = _sc_agg_scalar(z.reshape(NP), srcI, dstI, dinvp.reshape(NP),
                         self2.reshape(NP))
    return out[:N]
